# baseline (speedup 1.0000x reference)
import jax
import jax.numpy as jnp
from jax.experimental import pallas as pl
from jax.experimental.pallas import tpu as pltpu

ROWS = 8
DEPTH = 4


def _ffn_kernel(nu_ref, des_ref, rs_ref, ng_ref, tok_ref, valid_ref,
                x_ref, w1_hbm, w2_hbm, b1_ref, b2_ref, out_ref,
                wbuf, sem, _unused_sem):
    nu = nu_ref[0]
    D = w1_hbm.shape[1]

    def start_copy(v):
        e = des_ref[v]
        slot = jax.lax.rem(v, DEPTH)
        pltpu.make_async_copy(w1_hbm.at[e], wbuf.at[slot, pl.ds(0, D)],
                              sem.at[slot]).start()
        pltpu.make_async_copy(w2_hbm.at[e], wbuf.at[slot, pl.ds(D, D)],
                              sem.at[slot]).start()

    for d in range(DEPTH):
        @pl.when(d < nu)
        def _():
            start_copy(d)

    def expert_body(u, carry):
        slot = jax.lax.rem(u, DEPTH)
        e = des_ref[u]
        pltpu.make_async_copy(wbuf.at[0], wbuf.at[slot], sem.at[slot]).wait()

        b1row = b1_ref[pl.ds(e, 1), :]
        b2row = b2_ref[pl.ds(e, 1), :]
        base = rs_ref[u]

        def grp(j, c):
            p = (base + j) * ROWS
            rows = [x_ref[pl.ds(tok_ref[p + i], 1), :] for i in range(ROWS)]
            xb = jnp.concatenate(rows, axis=0)
            h = jnp.dot(xb, wbuf[slot, 0:D], preferred_element_type=jnp.float32)
            h = h + b1row
            h = 0.5 * h * (1.0 + jax.lax.erf(h * 0.7071067811865476))
            o = jnp.dot(h, wbuf[slot, D:2 * D], preferred_element_type=jnp.float32)
            o = o + b2row
            for i in range(ROWS):
                @pl.when(valid_ref[p + i] == 1)
                def _():
                    out_ref[pl.ds(tok_ref[p + i], 1), :] = o[i:i + 1, :]
            return c

        jax.lax.fori_loop(0, ng_ref[u], grp, 0, unroll=False)

        @pl.when(u + DEPTH < nu)
        def _():
            start_copy(u + DEPTH)
        return carry

    jax.lax.fori_loop(0, nu, expert_body, 0, unroll=False)


def kernel(x, indices_s, weight1, weight2, bias1, bias2):
    T, D = x.shape
    E, _, H = weight1.shape
    NB = T // ROWS + E

    idx = indices_s.astype(jnp.int32)
    sort_tok = jnp.argsort(idx, stable=True).astype(jnp.int32)
    sorted_e = idx[sort_tok]
    counts = jnp.bincount(idx, length=E)
    nb = (counts + ROWS - 1) // ROWS
    bend = jnp.cumsum(nb)
    bstart = bend - nb
    cstart = jnp.cumsum(counts) - counts
    rank = jnp.arange(T, dtype=jnp.int32) - cstart[sorted_e].astype(jnp.int32)
    pos = bstart[sorted_e].astype(jnp.int32) * ROWS + rank
    tok_at = jnp.zeros((NB * ROWS,), jnp.int32).at[pos].set(sort_tok)
    valid = jnp.zeros((NB * ROWS,), jnp.int32).at[pos].set(1)
    ids = jnp.arange(E, dtype=jnp.int32)
    key = jnp.where(counts > 0, ids, E + ids)
    des = jnp.argsort(key).astype(jnp.int32)
    nu = jnp.sum(counts > 0).astype(jnp.int32)[None]
    rs = bstart[des].astype(jnp.int32)
    ng = nb[des].astype(jnp.int32)

    grid_spec = pltpu.PrefetchScalarGridSpec(
        num_scalar_prefetch=6,
        grid=(1,),
        in_specs=[
            pl.BlockSpec((T, D), lambda u, *refs: (0, 0)),
            pl.BlockSpec(memory_space=pl.ANY),
            pl.BlockSpec(memory_space=pl.ANY),
            pl.BlockSpec((E, H), lambda u, *refs: (0, 0)),
            pl.BlockSpec((E, D), lambda u, *refs: (0, 0)),
        ],
        out_specs=pl.BlockSpec((T, D), lambda u, *refs: (0, 0)),
        scratch_shapes=[
            pltpu.VMEM((DEPTH, D + H, D), jnp.float32),
            pltpu.SemaphoreType.DMA((DEPTH,)),
            pltpu.SemaphoreType.DMA((1,)),
        ],
    )
    out = pl.pallas_call(
        _ffn_kernel,
        grid_spec=grid_spec,
        out_shape=jax.ShapeDtypeStruct((T, D), jnp.float32),
    )(nu, des, rs, ng, tok_at, valid, x, weight1, weight2, bias1, bias2)
    return out[:, None, :]


# speedup vs baseline: 1.4120x; 1.4120x over previous
import jax
import jax.numpy as jnp
from jax.experimental import pallas as pl
from jax.experimental.pallas import tpu as pltpu

ROWS = 8
DEPTH = 4


def _ffn_kernel(idx_ref, x_ref, w1_hbm, w2_hbm, b1_ref, b2_ref, out_ref,
                wbuf, sem, counts, basegrp, wp, des, rsu, ngu, cntu, tok):
    T = x_ref.shape[0]
    E = b1_ref.shape[0]
    D = w1_hbm.shape[1]

    def zero_body(e, c):
        counts[e] = 0
        return c
    jax.lax.fori_loop(0, E, zero_body, 0, unroll=False)

    def count_body(t, c):
        e = idx_ref[t]
        counts[e] = counts[e] + 1
        return c
    jax.lax.fori_loop(0, T, count_body, 0, unroll=False)

    def scan_body(e, carry):
        gacc, q = carry
        c = counts[e]
        ng = (c + ROWS - 1) // ROWS
        basegrp[e] = gacc
        wp[e] = 0

        @pl.when(c > 0)
        def _():
            des[q] = e
            rsu[q] = gacc
            ngu[q] = ng
            cntu[q] = c
        return gacc + ng, jnp.where(c > 0, q + 1, q)

    _, nu = jax.lax.fori_loop(0, E, scan_body,
                              (jnp.int32(0), jnp.int32(0)), unroll=False)

    def start_copy(v):
        e = des[v]
        slot = jax.lax.rem(v, DEPTH)
        pltpu.make_async_copy(w1_hbm.at[e], wbuf.at[slot, pl.ds(0, D)],
                              sem.at[slot]).start()
        pltpu.make_async_copy(w2_hbm.at[e], wbuf.at[slot, pl.ds(D, D)],
                              sem.at[slot]).start()

    for d in range(DEPTH):
        @pl.when(d < nu)
        def _():
            start_copy(d)

    def fill_body(t, c):
        e = idx_ref[t]
        w = wp[e]
        tok[basegrp[e] * ROWS + w] = t
        wp[e] = w + 1
        return c
    jax.lax.fori_loop(0, T, fill_body, 0, unroll=False)

    def expert_body(u, carry):
        slot = jax.lax.rem(u, DEPTH)
        e = des[u]
        pltpu.make_async_copy(wbuf.at[0], wbuf.at[slot], sem.at[slot]).wait()

        b1row = b1_ref[pl.ds(e, 1), :]
        b2row = b2_ref[pl.ds(e, 1), :]
        base = rsu[u]
        cnt = cntu[u]

        def grp(j, c):
            p = (base + j) * ROWS
            rem = cnt - j * ROWS
            tids = [jnp.clip(tok[p + i], 0, T - 1) for i in range(ROWS)]
            rows = [x_ref[pl.ds(tids[i], 1), :] for i in range(ROWS)]
            xb = jnp.concatenate(rows, axis=0)
            h = jnp.dot(xb, wbuf[slot, 0:D],
                        preferred_element_type=jnp.float32)
            h = h + b1row
            h = 0.5 * h * (1.0 + jax.lax.erf(h * 0.7071067811865476))
            o = jnp.dot(h, wbuf[slot, D:2 * D],
                        preferred_element_type=jnp.float32)
            o = o + b2row
            for i in range(ROWS):
                @pl.when(rem > i)
                def _():
                    out_ref[pl.ds(tids[i], 1), :] = o[i:i + 1, :]
            return c

        jax.lax.fori_loop(0, ngu[u], grp, 0, unroll=False)

        @pl.when(u + DEPTH < nu)
        def _():
            start_copy(u + DEPTH)
        return carry

    jax.lax.fori_loop(0, nu, expert_body, 0, unroll=False)


def kernel(x, indices_s, weight1, weight2, bias1, bias2):
    T, D = x.shape
    E, _, H = weight1.shape
    NB = T // ROWS + E

    idx = indices_s.astype(jnp.int32)

    grid_spec = pltpu.PrefetchScalarGridSpec(
        num_scalar_prefetch=1,
        grid=(1,),
        in_specs=[
            pl.BlockSpec((T, D), lambda u, *refs: (0, 0)),
            pl.BlockSpec(memory_space=pl.ANY),
            pl.BlockSpec(memory_space=pl.ANY),
            pl.BlockSpec((E, H), lambda u, *refs: (0, 0)),
            pl.BlockSpec((E, D), lambda u, *refs: (0, 0)),
        ],
        out_specs=pl.BlockSpec((T, D), lambda u, *refs: (0, 0)),
        scratch_shapes=[
            pltpu.VMEM((DEPTH, D + H, D), jnp.float32),
            pltpu.SemaphoreType.DMA((DEPTH,)),
            pltpu.SMEM((E,), jnp.int32),
            pltpu.SMEM((E,), jnp.int32),
            pltpu.SMEM((E,), jnp.int32),
            pltpu.SMEM((E,), jnp.int32),
            pltpu.SMEM((E,), jnp.int32),
            pltpu.SMEM((E,), jnp.int32),
            pltpu.SMEM((E,), jnp.int32),
            pltpu.SMEM((NB * ROWS,), jnp.int32),
        ],
    )
    out = pl.pallas_call(
        _ffn_kernel,
        grid_spec=grid_spec,
        out_shape=jax.ShapeDtypeStruct((T, D), jnp.float32),
    )(idx, x, weight1, weight2, bias1, bias2)
    return out[:, None, :]
